# baseline (speedup 1.0000x reference)
import functools

import jax
import jax.numpy as jnp
from jax import lax
from jax.experimental import pallas as pl
from jax.experimental.pallas import tpu as pltpu
from jax.experimental.pallas import tpu_sc as plsc

N_NODES = 10000
N_EDGES = 5000
N_EDGES_SL = 15000
NNZ_T = 330000
D = 128

N_PAD = 10240
E_PAD = 15360
E_HALF = E_PAD // 2
NW = 32
C = 128
CH = 82
CAP = NW * CH * C
RB = 512


def _sc_mesh():
    return plsc.VectorSubcoreMesh(core_axis_name="c", subcore_axis_name="s")


@functools.lru_cache(maxsize=None)
def _make_seg_kernel(dump_rows, dst_per_core):
    sr = dump_rows // 16
    chunks = 2 * CH if dst_per_core else CH

    @functools.partial(
        pl.kernel,
        out_type=jax.ShapeDtypeStruct((2, dump_rows, D), jnp.float32),
        mesh=_sc_mesh(),
        scratch_types=[
            pltpu.VMEM((C,), jnp.int32),
            pltpu.VMEM((C,), jnp.int32),
            pltpu.VMEM((C, D), jnp.float32),
            pltpu.VMEM_SHARED((dump_rows + 8, D), jnp.float32),
            pltpu.SemaphoreType.DMA,
        ],
    )
    def seg(src_hbm, sidx_hbm, didx_hbm, zeros_hbm, out_hbm,
            sidx_v, didx_v, rows_v, acc, sem):
        c = lax.axis_index("c")
        s = lax.axis_index("s")
        wid = c * 16 + s
        pltpu.sync_copy(zeros_hbm.at[pl.ds(0, sr)], acc.at[pl.ds(s * sr, sr)])

        @pl.when(s == 0)
        def _():
            pltpu.sync_copy(zeros_hbm.at[pl.ds(0, 8)], acc.at[pl.ds(dump_rows, 8)])

        plsc.subcore_barrier()

        def body(j, carry):
            pltpu.sync_copy(sidx_hbm.at[wid, j], sidx_v)
            pltpu.sync_copy(didx_hbm.at[wid, j], didx_v)
            pltpu.async_copy(src_hbm.at[sidx_v], rows_v, sem).wait()
            pltpu.sync_copy(rows_v, acc.at[didx_v], add=True)
            return carry

        lax.fori_loop(0, chunks, body, 0)
        plsc.subcore_barrier()
        pltpu.sync_copy(acc.at[pl.ds(s * sr, sr)],
                        out_hbm.at[c, pl.ds(s * sr, sr)])

    return seg


@functools.lru_cache(maxsize=None)
def _make_deg_kernel(dump_rows, dst_per_core):
    sr = dump_rows // 16
    chunks = 2 * CH if dst_per_core else CH

    @functools.partial(
        pl.kernel,
        out_type=jax.ShapeDtypeStruct((2, dump_rows, D), jnp.float32),
        mesh=_sc_mesh(),
        scratch_types=[
            pltpu.VMEM((C,), jnp.int32),
            pltpu.VMEM((C, D), jnp.float32),
            pltpu.VMEM_SHARED((dump_rows + 8, D), jnp.float32),
        ],
    )
    def deg(didx_hbm, ones_hbm, zeros_hbm, out_hbm, didx_v, ones_v, acc):
        c = lax.axis_index("c")
        s = lax.axis_index("s")
        wid = c * 16 + s
        pltpu.sync_copy(ones_hbm, ones_v)
        pltpu.sync_copy(zeros_hbm.at[pl.ds(0, sr)], acc.at[pl.ds(s * sr, sr)])

        @pl.when(s == 0)
        def _():
            pltpu.sync_copy(zeros_hbm.at[pl.ds(0, 8)], acc.at[pl.ds(dump_rows, 8)])

        plsc.subcore_barrier()

        def body(j, carry):
            pltpu.sync_copy(didx_hbm.at[wid, j], didx_v)
            pltpu.sync_copy(ones_v, acc.at[didx_v], add=True)
            return carry

        lax.fori_loop(0, chunks, body, 0)
        plsc.subcore_barrier()
        pltpu.sync_copy(acc.at[pl.ds(s * sr, sr)],
                        out_hbm.at[c, pl.ds(s * sr, sr)])

    return deg


@functools.lru_cache(maxsize=None)
def _make_mm(n_rows):
    def body(x_ref, w_ref, y_ref):
        y_ref[...] = jnp.dot(x_ref[...], w_ref[...],
                             preferred_element_type=jnp.float32)

    return pl.pallas_call(
        body,
        grid=(n_rows // RB,),
        in_specs=[
            pl.BlockSpec((RB, D), lambda i: (i, 0)),
            pl.BlockSpec((D, D), lambda i: (0, 0)),
        ],
        out_specs=pl.BlockSpec((RB, D), lambda i: (i, 0)),
        out_shape=jax.ShapeDtypeStruct((n_rows, D), jnp.float32),
    )


@functools.lru_cache(maxsize=None)
def _make_combine(n_rows, two_p, with_ev, with_mm):
    def body(*refs):
        refs = list(refs)
        k = 2 if two_p else 1
        p = refs[0][...]
        cnt = refs[k][...][:, 0:1]
        if two_p:
            p = p + refs[1][...]
            cnt = cnt + refs[k + 1][...][:, 0:1]
            k += 2
        else:
            k += 1
        b_ref, a_ref = refs[k:k + 2]
        k += 2
        ev_ref = w_ref = None
        if with_ev:
            ev_ref = refs[k]; k += 1
        if with_mm:
            w_ref = refs[k]; k += 1
        act_ref = refs[k]; k += 1
        y_ref = refs[k] if with_mm else None

        scale = jnp.where(cnt > 0, 1.0 / jnp.where(cnt > 0, cnt, 1.0), 0.0)
        pre = p * scale + b_ref[0:1, :]
        if with_ev:
            pre = pre + ev_ref[...]
        a = a_ref[0, 0]
        act = jnp.where(pre >= 0, pre, a * pre)
        act_ref[...] = act
        if with_mm:
            y_ref[...] = jnp.dot(act, w_ref[...],
                                 preferred_element_type=jnp.float32)

    blk = pl.BlockSpec((RB, D), lambda i: (i, 0))
    in_specs = [blk] * (2 if two_p else 1)
    in_specs += [blk] * (2 if two_p else 1)
    in_specs += [
        pl.BlockSpec((8, D), lambda i: (0, 0)),
        pl.BlockSpec(memory_space=pltpu.SMEM),
    ]
    if with_ev:
        in_specs.append(blk)
    if with_mm:
        in_specs.append(pl.BlockSpec((D, D), lambda i: (0, 0)))
    out_specs = [blk]
    out_shape = [jax.ShapeDtypeStruct((n_rows, D), jnp.float32)]
    if with_mm:
        out_specs.append(blk)
        out_shape.append(jax.ShapeDtypeStruct((n_rows, D), jnp.float32))

    return pl.pallas_call(
        body,
        grid=(n_rows // RB,),
        in_specs=in_specs,
        out_specs=out_specs,
        out_shape=out_shape,
    )


def _combine(parts, cnts, b, a, ev=None, w=None):
    parts = tuple(parts)
    cnts = tuple(cnts)
    assert len(parts) == len(cnts)
    n_rows = parts[0].shape[0]
    args = list(parts) + list(cnts)
    args += [jnp.broadcast_to(b[None, :], (8, D)), jnp.reshape(a, (1, 1))]
    if ev is not None:
        args.append(ev)
    if w is not None:
        args.append(w)
    out = _make_combine(n_rows, len(parts) == 2, ev is not None,
                        w is not None)(*args)
    return (out[0], out[1]) if w is not None else (out[0], None)


def kernel(x, xx, params, hyperedge_index, num_nodes, num_edges):
    del num_nodes, num_edges
    idx_n = jnp.concatenate(
        [hyperedge_index[0], jnp.arange(N_NODES, dtype=jnp.int32)])
    idx_e = jnp.concatenate(
        [hyperedge_index[1],
         jnp.arange(N_EDGES, N_EDGES_SL, dtype=jnp.int32)])
    pad = CAP - NNZ_T
    esrc = jnp.pad(idx_e, (0, pad)).reshape(NW, CH, C)
    ndst = jnp.pad(idx_n, (0, pad), constant_values=N_PAD).reshape(NW, CH, C)
    nsrc_s = jnp.pad(idx_n, (0, pad)).reshape(16, 2 * CH, C)
    edst_s = jnp.pad(idx_e, (0, pad),
                     constant_values=E_PAD).reshape(16, 2 * CH, C)
    edst0 = jnp.where(edst_s < E_HALF, edst_s, E_HALF)
    e1l = edst_s - E_HALF
    edst1 = jnp.where((e1l >= 0) & (e1l < E_HALF), e1l, E_HALF)
    nsrc_b = jnp.concatenate([nsrc_s, nsrc_s], axis=0)
    edst_b = jnp.concatenate([edst0, edst1], axis=0)

    zeros = jnp.zeros((E_PAD // 16, D), jnp.float32)
    ones = jnp.ones((C, D), jnp.float32)

    cnt_n_p = _make_deg_kernel(N_PAD, False)(ndst, ones, zeros)
    cnt_e_p = _make_deg_kernel(E_HALF, True)(edst_b, ones, zeros)
    cnt_n = (cnt_n_p[0], cnt_n_p[1])
    cnt_e = (cnt_e_p.reshape(E_PAD, D),)

    x_p = jnp.pad(x, ((0, N_PAD - N_NODES), (0, 0)))
    xx_p = jnp.pad(xx, ((0, N_PAD - N_NODES), (0, 0)))

    seg_n2e = _make_seg_kernel(E_HALF, True)
    seg_e2n = _make_seg_kernel(N_PAD, False)

    def n2e(xw):
        return (seg_n2e(xw, nsrc_b, edst_b, zeros).reshape(E_PAD, D),)

    def e2n(xn):
        out = seg_e2n(xn, esrc, ndst, zeros)
        return (out[0], out[1])

    def encoder(xv, g):
        a = g["a"]
        xw = _make_mm(N_PAD)(xv, g["Wn2e"][0])
        _, xn = _combine(n2e(xw), cnt_e, g["bn2e"][0], a, w=g["We2n"][0])
        _, xw = _combine(e2n(xn), cnt_n, g["be2n"][0], a, w=g["Wn2e"][1])
        e_act, xn = _combine(n2e(xw), cnt_e, g["bn2e"][1], a, w=g["We2n"][1])
        n_act, _ = _combine(e2n(xn), cnt_n, g["be2n"][1], a)
        return n_act, e_act

    def decoder(nv, ev, g):
        a = g["a"]
        xw = _make_mm(N_PAD)(nv, g["Wn2e"][0])
        _, xn = _combine(n2e(xw), cnt_e, g["bn2e"][0], a, ev=ev, w=g["We2n"][0])
        _, xw = _combine(e2n(xn), cnt_n, g["be2n"][0], a, w=g["Wn2e"][1])
        _, xn = _combine(n2e(xw), cnt_e, g["bn2e"][1], a, ev=ev, w=g["We2n"][1])
        n_act, _ = _combine(e2n(xn), cnt_n, g["be2n"][1], a)
        return n_act

    n1p, e1p = encoder(x_p, params["enc1"])
    n2p, e2p = encoder(xx_p, params["enc2"])
    x11 = decoder(n1p, e1p, params["dec1"])
    x21 = decoder(n2p, e2p, params["dec2"])
    x12 = decoder(n2p, e2p, params["dec1"])
    x22 = decoder(n1p, e1p, params["dec2"])

    n1 = n1p[:N_NODES]
    n2 = n2p[:N_NODES]
    nn1 = jnp.concatenate([n1, n2], axis=1)
    return (nn1, n1, e1p[:N_EDGES], n2, e2p[:N_EDGES],
            x11[:N_NODES], x21[:N_NODES], x12[:N_NODES], x22[:N_NODES])


# speedup vs baseline: 1.4683x; 1.4683x over previous
import functools

import jax
import jax.numpy as jnp
from jax import lax
from jax.experimental import pallas as pl
from jax.experimental.pallas import tpu as pltpu
from jax.experimental.pallas import tpu_sc as plsc

N_NODES = 10000
N_EDGES = 5000
N_EDGES_SL = 15000
NNZ_T = 330000
D = 128

N_PAD = 10240
E_PAD = 15360
E_HALF = E_PAD // 2
NW = 32
C = 128
CH = 81
CAP = NW * CH * C
RB = 512


def _sc_mesh():
    return plsc.VectorSubcoreMesh(core_axis_name="c", subcore_axis_name="s")


@functools.lru_cache(maxsize=None)
def _make_seg_kernel(dump_rows, dst_per_core):
    sr = dump_rows // 16
    chunks = 2 * CH if dst_per_core else CH

    @functools.partial(
        pl.kernel,
        out_type=jax.ShapeDtypeStruct((2, dump_rows, D), jnp.float32),
        mesh=_sc_mesh(),
        scratch_types=[
            pltpu.VMEM((C,), jnp.int32),
            pltpu.VMEM((C,), jnp.int32),
            pltpu.VMEM((C, D), jnp.float32),
            pltpu.VMEM_SHARED((dump_rows + 8, D), jnp.float32),
            pltpu.SemaphoreType.DMA,
        ],
    )
    def seg(src_hbm, sidx_hbm, didx_hbm, zeros_hbm, out_hbm,
            sidx_v, didx_v, rows_v, acc, sem):
        c = lax.axis_index("c")
        s = lax.axis_index("s")
        wid = c * 16 + s
        pltpu.sync_copy(zeros_hbm.at[pl.ds(0, sr)], acc.at[pl.ds(s * sr, sr)])

        @pl.when(s == 0)
        def _():
            pltpu.sync_copy(zeros_hbm.at[pl.ds(0, 8)], acc.at[pl.ds(dump_rows, 8)])

        plsc.subcore_barrier()

        def body(j, carry):
            if dst_per_core:
                pltpu.sync_copy(sidx_hbm.at[s, j], sidx_v)
            else:
                pltpu.sync_copy(sidx_hbm.at[wid, j], sidx_v)
            pltpu.sync_copy(didx_hbm.at[wid, j], didx_v)
            pltpu.async_copy(src_hbm.at[sidx_v], rows_v, sem).wait()
            pltpu.sync_copy(rows_v, acc.at[didx_v], add=True)
            return carry

        lax.fori_loop(0, chunks, body, 0)
        plsc.subcore_barrier()
        pltpu.sync_copy(acc.at[pl.ds(s * sr, sr)],
                        out_hbm.at[c, pl.ds(s * sr, sr)])

    return seg


@functools.lru_cache(maxsize=None)
def _make_deg_kernel(dump_rows, dst_per_core):
    sr = dump_rows // 16
    chunks = 2 * CH if dst_per_core else CH

    @functools.partial(
        pl.kernel,
        out_type=jax.ShapeDtypeStruct((2, dump_rows, D), jnp.float32),
        mesh=_sc_mesh(),
        scratch_types=[
            pltpu.VMEM((C,), jnp.int32),
            pltpu.VMEM((C, D), jnp.float32),
            pltpu.VMEM_SHARED((dump_rows + 8, D), jnp.float32),
        ],
    )
    def deg(didx_hbm, ones_hbm, zeros_hbm, out_hbm, didx_v, ones_v, acc):
        c = lax.axis_index("c")
        s = lax.axis_index("s")
        wid = c * 16 + s
        pltpu.sync_copy(ones_hbm, ones_v)
        pltpu.sync_copy(zeros_hbm.at[pl.ds(0, sr)], acc.at[pl.ds(s * sr, sr)])

        @pl.when(s == 0)
        def _():
            pltpu.sync_copy(zeros_hbm.at[pl.ds(0, 8)], acc.at[pl.ds(dump_rows, 8)])

        plsc.subcore_barrier()

        def body(j, carry):
            pltpu.sync_copy(didx_hbm.at[wid, j], didx_v)
            pltpu.sync_copy(ones_v, acc.at[didx_v], add=True)
            return carry

        lax.fori_loop(0, chunks, body, 0)
        plsc.subcore_barrier()
        pltpu.sync_copy(acc.at[pl.ds(s * sr, sr)],
                        out_hbm.at[c, pl.ds(s * sr, sr)])

    return deg


@functools.lru_cache(maxsize=None)
def _make_mm(n_rows):
    def body(x_ref, w_ref, y_ref):
        y_ref[...] = jnp.dot(x_ref[...], w_ref[...],
                             preferred_element_type=jnp.float32)

    return pl.pallas_call(
        body,
        grid=(n_rows // RB,),
        in_specs=[
            pl.BlockSpec((RB, D), lambda i: (i, 0)),
            pl.BlockSpec((D, D), lambda i: (0, 0)),
        ],
        out_specs=pl.BlockSpec((RB, D), lambda i: (i, 0)),
        out_shape=jax.ShapeDtypeStruct((n_rows, D), jnp.float32),
    )


@functools.lru_cache(maxsize=None)
def _make_combine(n_rows, two_p, with_ev, with_mm):
    def body(*refs):
        refs = list(refs)
        k = 2 if two_p else 1
        p = refs[0][...]
        cnt = refs[k][...][:, 0:1]
        if two_p:
            p = p + refs[1][...]
            cnt = cnt + refs[k + 1][...][:, 0:1]
            k += 2
        else:
            k += 1
        b_ref, a_ref = refs[k:k + 2]
        k += 2
        ev_ref = w_ref = None
        if with_ev:
            ev_ref = refs[k]; k += 1
        if with_mm:
            w_ref = refs[k]; k += 1
        act_ref = refs[k]; k += 1
        y_ref = refs[k] if with_mm else None

        scale = jnp.where(cnt > 0, 1.0 / jnp.where(cnt > 0, cnt, 1.0), 0.0)
        pre = p * scale + b_ref[0:1, :]
        if with_ev:
            pre = pre + ev_ref[...]
        a = a_ref[0, 0]
        act = jnp.where(pre >= 0, pre, a * pre)
        act_ref[...] = act
        if with_mm:
            y_ref[...] = jnp.dot(act, w_ref[...],
                                 preferred_element_type=jnp.float32)

    blk = pl.BlockSpec((RB, D), lambda i: (i, 0))
    in_specs = [blk] * (2 if two_p else 1)
    in_specs += [blk] * (2 if two_p else 1)
    in_specs += [
        pl.BlockSpec((8, D), lambda i: (0, 0)),
        pl.BlockSpec(memory_space=pltpu.SMEM),
    ]
    if with_ev:
        in_specs.append(blk)
    if with_mm:
        in_specs.append(pl.BlockSpec((D, D), lambda i: (0, 0)))
    out_specs = [blk]
    out_shape = [jax.ShapeDtypeStruct((n_rows, D), jnp.float32)]
    if with_mm:
        out_specs.append(blk)
        out_shape.append(jax.ShapeDtypeStruct((n_rows, D), jnp.float32))

    return pl.pallas_call(
        body,
        grid=(n_rows // RB,),
        in_specs=in_specs,
        out_specs=out_specs,
        out_shape=out_shape,
    )


def _combine(parts, cnts, b, a, ev=None, w=None):
    parts = tuple(parts)
    cnts = tuple(cnts)
    assert len(parts) == len(cnts)
    n_rows = parts[0].shape[0]
    args = list(parts) + list(cnts)
    args += [jnp.broadcast_to(b[None, :], (8, D)), jnp.reshape(a, (1, 1))]
    if ev is not None:
        args.append(ev)
    if w is not None:
        args.append(w)
    out = _make_combine(n_rows, len(parts) == 2, ev is not None,
                        w is not None)(*args)
    return (out[0], out[1]) if w is not None else (out[0], None)


def kernel(x, xx, params, hyperedge_index, num_nodes, num_edges):
    del num_nodes, num_edges
    idx_n = jnp.concatenate(
        [hyperedge_index[0], jnp.arange(N_NODES, dtype=jnp.int32)])
    idx_e = jnp.concatenate(
        [hyperedge_index[1],
         jnp.arange(N_EDGES, N_EDGES_SL, dtype=jnp.int32)])
    pad = CAP - NNZ_T
    esrc = jnp.pad(idx_e, (0, pad)).reshape(NW, CH, C)
    ndst = jnp.pad(idx_n, (0, pad), constant_values=N_PAD).reshape(NW, CH, C)
    nsrc_s = jnp.pad(idx_n, (0, pad)).reshape(16, 2 * CH, C)
    edst_s = jnp.pad(idx_e, (0, pad),
                     constant_values=E_PAD).reshape(16, 2 * CH, C)
    edst0 = jnp.where(edst_s < E_HALF, edst_s, E_HALF)
    e1l = edst_s - E_HALF
    edst1 = jnp.where((e1l >= 0) & (e1l < E_HALF), e1l, E_HALF)
    edst_b = jnp.concatenate([edst0, edst1], axis=0)

    zeros = jnp.zeros((E_PAD // 16, D), jnp.float32)
    ones = jnp.ones((C, D), jnp.float32)

    cnt_n_p = _make_deg_kernel(N_PAD, False)(ndst, ones, zeros)
    cnt_e_p = _make_deg_kernel(E_HALF, True)(edst_b, ones, zeros)
    cnt_n = (cnt_n_p[0], cnt_n_p[1])
    cnt_e = (cnt_e_p.reshape(E_PAD, D),)

    x_p = jnp.pad(x, ((0, N_PAD - N_NODES), (0, 0)))
    xx_p = jnp.pad(xx, ((0, N_PAD - N_NODES), (0, 0)))

    seg_n2e = _make_seg_kernel(E_HALF, True)
    seg_e2n = _make_seg_kernel(N_PAD, False)

    def n2e(xw):
        return (seg_n2e(xw, nsrc_s, edst_b, zeros).reshape(E_PAD, D),)

    def e2n(xn):
        out = seg_e2n(xn, esrc, ndst, zeros)
        return (out[0], out[1])

    def encoder(xv, g):
        a = g["a"]
        xw = _make_mm(N_PAD)(xv, g["Wn2e"][0])
        _, xn = _combine(n2e(xw), cnt_e, g["bn2e"][0], a, w=g["We2n"][0])
        _, xw = _combine(e2n(xn), cnt_n, g["be2n"][0], a, w=g["Wn2e"][1])
        e_act, xn = _combine(n2e(xw), cnt_e, g["bn2e"][1], a, w=g["We2n"][1])
        n_act, _ = _combine(e2n(xn), cnt_n, g["be2n"][1], a)
        return n_act, e_act

    def decoder(nv, ev, g):
        a = g["a"]
        xw = _make_mm(N_PAD)(nv, g["Wn2e"][0])
        _, xn = _combine(n2e(xw), cnt_e, g["bn2e"][0], a, ev=ev, w=g["We2n"][0])
        _, xw = _combine(e2n(xn), cnt_n, g["be2n"][0], a, w=g["Wn2e"][1])
        _, xn = _combine(n2e(xw), cnt_e, g["bn2e"][1], a, ev=ev, w=g["We2n"][1])
        n_act, _ = _combine(e2n(xn), cnt_n, g["be2n"][1], a)
        return n_act

    n1p, e1p = encoder(x_p, params["enc1"])
    n2p, e2p = encoder(xx_p, params["enc2"])
    x11 = decoder(n1p, e1p, params["dec1"])
    x21 = decoder(n2p, e2p, params["dec2"])
    x12 = decoder(n2p, e2p, params["dec1"])
    x22 = decoder(n1p, e1p, params["dec2"])

    n1 = n1p[:N_NODES]
    n2 = n2p[:N_NODES]
    nn1 = jnp.concatenate([n1, n2], axis=1)
    return (nn1, n1, e1p[:N_EDGES], n2, e2p[:N_EDGES],
            x11[:N_NODES], x21[:N_NODES], x12[:N_NODES], x22[:N_NODES])


# speedup vs baseline: 1.7456x; 1.1889x over previous
import functools

import jax
import jax.numpy as jnp
from jax import lax
from jax.experimental import pallas as pl
from jax.experimental.pallas import tpu as pltpu
from jax.experimental.pallas import tpu_sc as plsc

N_NODES = 10000
N_EDGES = 5000
N_EDGES_SL = 15000
NNZ_T = 330000
D = 128

N_PAD = 10240
E_PAD = 15360
E_HALF = E_PAD // 2
NW = 32
C = 128
CH = 81
CAP = NW * CH * C
RB = 512


def _sc_mesh():
    return plsc.VectorSubcoreMesh(core_axis_name="c", subcore_axis_name="s")


@functools.lru_cache(maxsize=None)
def _make_seg_kernel(dump_rows, dst_per_core):
    sr = dump_rows // 16
    chunks = 2 * CH if dst_per_core else CH

    @functools.partial(
        pl.kernel,
        out_type=jax.ShapeDtypeStruct((2, dump_rows, D), jnp.float32),
        mesh=_sc_mesh(),
        scratch_types=[
            pltpu.VMEM((2 * CH if dst_per_core else CH, C), jnp.int32),
            pltpu.VMEM((2 * CH if dst_per_core else CH, C), jnp.int32),
            pltpu.VMEM((C, D), jnp.float32),
            pltpu.VMEM_SHARED((dump_rows + 8, D), jnp.float32),
            pltpu.SemaphoreType.DMA,
        ],
    )
    def seg(src_hbm, sidx_hbm, didx_hbm, zeros_hbm, out_hbm,
            sidx_v, didx_v, rows_v, acc, sem):
        c = lax.axis_index("c")
        s = lax.axis_index("s")
        wid = c * 16 + s
        pltpu.sync_copy(zeros_hbm.at[pl.ds(0, sr)], acc.at[pl.ds(s * sr, sr)])

        @pl.when(s == 0)
        def _():
            pltpu.sync_copy(zeros_hbm.at[pl.ds(0, 8)], acc.at[pl.ds(dump_rows, 8)])

        if dst_per_core:
            pltpu.sync_copy(sidx_hbm.at[s], sidx_v)
        else:
            pltpu.sync_copy(sidx_hbm.at[wid], sidx_v)
        pltpu.sync_copy(didx_hbm.at[wid], didx_v)
        plsc.subcore_barrier()

        def body(j, carry):
            pltpu.async_copy(src_hbm.at[sidx_v.at[j]], rows_v, sem).wait()
            pltpu.sync_copy(rows_v, acc.at[didx_v.at[j]], add=True)
            return carry

        lax.fori_loop(0, chunks, body, 0)
        plsc.subcore_barrier()
        pltpu.sync_copy(acc.at[pl.ds(s * sr, sr)],
                        out_hbm.at[c, pl.ds(s * sr, sr)])

    return seg


@functools.lru_cache(maxsize=None)
def _make_deg_kernel(dump_rows, dst_per_core):
    sr = dump_rows // 16
    chunks = 2 * CH if dst_per_core else CH

    @functools.partial(
        pl.kernel,
        out_type=jax.ShapeDtypeStruct((2, dump_rows, D), jnp.float32),
        mesh=_sc_mesh(),
        scratch_types=[
            pltpu.VMEM((2 * CH if dst_per_core else CH, C), jnp.int32),
            pltpu.VMEM((C, D), jnp.float32),
            pltpu.VMEM_SHARED((dump_rows + 8, D), jnp.float32),
        ],
    )
    def deg(didx_hbm, ones_hbm, zeros_hbm, out_hbm, didx_v, ones_v, acc):
        c = lax.axis_index("c")
        s = lax.axis_index("s")
        wid = c * 16 + s
        pltpu.sync_copy(ones_hbm, ones_v)
        pltpu.sync_copy(zeros_hbm.at[pl.ds(0, sr)], acc.at[pl.ds(s * sr, sr)])

        @pl.when(s == 0)
        def _():
            pltpu.sync_copy(zeros_hbm.at[pl.ds(0, 8)], acc.at[pl.ds(dump_rows, 8)])

        pltpu.sync_copy(didx_hbm.at[wid], didx_v)
        plsc.subcore_barrier()

        def body(j, carry):
            pltpu.sync_copy(ones_v, acc.at[didx_v.at[j]], add=True)
            return carry

        lax.fori_loop(0, chunks, body, 0)
        plsc.subcore_barrier()
        pltpu.sync_copy(acc.at[pl.ds(s * sr, sr)],
                        out_hbm.at[c, pl.ds(s * sr, sr)])

    return deg


@functools.lru_cache(maxsize=None)
def _make_mm(n_rows):
    def body(x_ref, w_ref, y_ref):
        y_ref[...] = jnp.dot(x_ref[...], w_ref[...],
                             preferred_element_type=jnp.float32)

    return pl.pallas_call(
        body,
        grid=(n_rows // RB,),
        in_specs=[
            pl.BlockSpec((RB, D), lambda i: (i, 0)),
            pl.BlockSpec((D, D), lambda i: (0, 0)),
        ],
        out_specs=pl.BlockSpec((RB, D), lambda i: (i, 0)),
        out_shape=jax.ShapeDtypeStruct((n_rows, D), jnp.float32),
    )


@functools.lru_cache(maxsize=None)
def _make_combine(n_rows, two_p, with_ev, with_mm):
    def body(*refs):
        refs = list(refs)
        k = 2 if two_p else 1
        p = refs[0][...]
        cnt = refs[k][...][:, 0:1]
        if two_p:
            p = p + refs[1][...]
            cnt = cnt + refs[k + 1][...][:, 0:1]
            k += 2
        else:
            k += 1
        b_ref, a_ref = refs[k:k + 2]
        k += 2
        ev_ref = w_ref = None
        if with_ev:
            ev_ref = refs[k]; k += 1
        if with_mm:
            w_ref = refs[k]; k += 1
        act_ref = refs[k]; k += 1
        y_ref = refs[k] if with_mm else None

        scale = jnp.where(cnt > 0, 1.0 / jnp.where(cnt > 0, cnt, 1.0), 0.0)
        pre = p * scale + b_ref[0:1, :]
        if with_ev:
            pre = pre + ev_ref[...]
        a = a_ref[0, 0]
        act = jnp.where(pre >= 0, pre, a * pre)
        act_ref[...] = act
        if with_mm:
            y_ref[...] = jnp.dot(act, w_ref[...],
                                 preferred_element_type=jnp.float32)

    blk = pl.BlockSpec((RB, D), lambda i: (i, 0))
    in_specs = [blk] * (2 if two_p else 1)
    in_specs += [blk] * (2 if two_p else 1)
    in_specs += [
        pl.BlockSpec((8, D), lambda i: (0, 0)),
        pl.BlockSpec(memory_space=pltpu.SMEM),
    ]
    if with_ev:
        in_specs.append(blk)
    if with_mm:
        in_specs.append(pl.BlockSpec((D, D), lambda i: (0, 0)))
    out_specs = [blk]
    out_shape = [jax.ShapeDtypeStruct((n_rows, D), jnp.float32)]
    if with_mm:
        out_specs.append(blk)
        out_shape.append(jax.ShapeDtypeStruct((n_rows, D), jnp.float32))

    return pl.pallas_call(
        body,
        grid=(n_rows // RB,),
        in_specs=in_specs,
        out_specs=out_specs,
        out_shape=out_shape,
    )


def _combine(parts, cnts, b, a, ev=None, w=None):
    parts = tuple(parts)
    cnts = tuple(cnts)
    assert len(parts) == len(cnts)
    n_rows = parts[0].shape[0]
    args = list(parts) + list(cnts)
    args += [jnp.broadcast_to(b[None, :], (8, D)), jnp.reshape(a, (1, 1))]
    if ev is not None:
        args.append(ev)
    if w is not None:
        args.append(w)
    out = _make_combine(n_rows, len(parts) == 2, ev is not None,
                        w is not None)(*args)
    return (out[0], out[1]) if w is not None else (out[0], None)


def kernel(x, xx, params, hyperedge_index, num_nodes, num_edges):
    del num_nodes, num_edges
    idx_n = jnp.concatenate(
        [hyperedge_index[0], jnp.arange(N_NODES, dtype=jnp.int32)])
    idx_e = jnp.concatenate(
        [hyperedge_index[1],
         jnp.arange(N_EDGES, N_EDGES_SL, dtype=jnp.int32)])
    pad = CAP - NNZ_T
    esrc = jnp.pad(idx_e, (0, pad)).reshape(NW, CH, C)
    ndst = jnp.pad(idx_n, (0, pad), constant_values=N_PAD).reshape(NW, CH, C)
    nsrc_s = jnp.pad(idx_n, (0, pad)).reshape(16, 2 * CH, C)
    edst_s = jnp.pad(idx_e, (0, pad),
                     constant_values=E_PAD).reshape(16, 2 * CH, C)
    edst0 = jnp.where(edst_s < E_HALF, edst_s, E_HALF)
    e1l = edst_s - E_HALF
    edst1 = jnp.where((e1l >= 0) & (e1l < E_HALF), e1l, E_HALF)
    edst_b = jnp.concatenate([edst0, edst1], axis=0)

    zeros = jnp.zeros((E_PAD // 16, D), jnp.float32)
    ones = jnp.ones((C, D), jnp.float32)

    cnt_n_p = _make_deg_kernel(N_PAD, False)(ndst, ones, zeros)
    cnt_e_p = _make_deg_kernel(E_HALF, True)(edst_b, ones, zeros)
    cnt_n = (cnt_n_p[0], cnt_n_p[1])
    cnt_e = (cnt_e_p.reshape(E_PAD, D),)

    x_p = jnp.pad(x, ((0, N_PAD - N_NODES), (0, 0)))
    xx_p = jnp.pad(xx, ((0, N_PAD - N_NODES), (0, 0)))

    seg_n2e = _make_seg_kernel(E_HALF, True)
    seg_e2n = _make_seg_kernel(N_PAD, False)

    def n2e(xw):
        return (seg_n2e(xw, nsrc_s, edst_b, zeros).reshape(E_PAD, D),)

    def e2n(xn):
        out = seg_e2n(xn, esrc, ndst, zeros)
        return (out[0], out[1])

    def encoder(xv, g):
        a = g["a"]
        xw = _make_mm(N_PAD)(xv, g["Wn2e"][0])
        _, xn = _combine(n2e(xw), cnt_e, g["bn2e"][0], a, w=g["We2n"][0])
        _, xw = _combine(e2n(xn), cnt_n, g["be2n"][0], a, w=g["Wn2e"][1])
        e_act, xn = _combine(n2e(xw), cnt_e, g["bn2e"][1], a, w=g["We2n"][1])
        n_act, _ = _combine(e2n(xn), cnt_n, g["be2n"][1], a)
        return n_act, e_act

    def decoder(nv, ev, g):
        a = g["a"]
        xw = _make_mm(N_PAD)(nv, g["Wn2e"][0])
        _, xn = _combine(n2e(xw), cnt_e, g["bn2e"][0], a, ev=ev, w=g["We2n"][0])
        _, xw = _combine(e2n(xn), cnt_n, g["be2n"][0], a, w=g["Wn2e"][1])
        _, xn = _combine(n2e(xw), cnt_e, g["bn2e"][1], a, ev=ev, w=g["We2n"][1])
        n_act, _ = _combine(e2n(xn), cnt_n, g["be2n"][1], a)
        return n_act

    n1p, e1p = encoder(x_p, params["enc1"])
    n2p, e2p = encoder(xx_p, params["enc2"])
    x11 = decoder(n1p, e1p, params["dec1"])
    x21 = decoder(n2p, e2p, params["dec2"])
    x12 = decoder(n2p, e2p, params["dec1"])
    x22 = decoder(n1p, e1p, params["dec2"])

    n1 = n1p[:N_NODES]
    n2 = n2p[:N_NODES]
    nn1 = jnp.concatenate([n1, n2], axis=1)
    return (nn1, n1, e1p[:N_EDGES], n2, e2p[:N_EDGES],
            x11[:N_NODES], x21[:N_NODES], x12[:N_NODES], x22[:N_NODES])
